# baseline (speedup 1.0000x reference)
import functools

import jax
import jax.numpy as jnp
from jax import lax
from jax.experimental import pallas as pl
from jax.experimental.pallas import tpu as pltpu
from jax.experimental.pallas import tpu_sc as plsc
from jax._src.pallas import mpmd
from jax._src.pallas.mosaic.core import MemorySpace as _MS

MAX_LEN = 8192
D_MODEL = 1024
NUM_CORES = 2
NUM_SUBCORES = 16
ROWS_PER_CORE = MAX_LEN // NUM_CORES

SCS_ROWS = 1664
SCS_CHUNK = 128
SCS_NCHUNK = SCS_ROWS // SCS_CHUNK

TEC_ROWS_PER_CORE = ROWS_PER_CORE - SCS_ROWS
TEC_ROWS = TEC_ROWS_PER_CORE // NUM_SUBCORES
TEC_CHUNK = 8
TEC_NCHUNK = TEC_ROWS // TEC_CHUNK

_scalar_mesh = plsc.ScalarSubcoreMesh(axis_name="c", num_cores=NUM_CORES)
_vector_mesh = plsc.VectorSubcoreMesh(core_axis_name="c", subcore_axis_name="s")


def _pipelined_copy(w_hbm, out_hbm, base, chunk, nchunk, bufs, sins, souts):
    def in_slice(c):
        return w_hbm.at[pl.ds(base + c * chunk, chunk)]

    def out_slice(c):
        return out_hbm.at[pl.ds(base + c * chunk, chunk)]

    hin = [None] * nchunk
    hout = [None] * nchunk
    hin[0] = pltpu.async_copy(in_slice(0), bufs[0], sins[0])
    for c in range(nchunk):
        if c + 1 < nchunk:
            b = (c + 1) % 2
            if c >= 1:
                hout[c - 1].wait()
            hin[c + 1] = pltpu.async_copy(in_slice(c + 1), bufs[b], sins[b])
        hin[c].wait()
        hout[c] = pltpu.async_copy(bufs[c % 2], out_slice(c), souts[c % 2])
    if nchunk >= 2:
        hout[nchunk - 2].wait()
    hout[nchunk - 1].wait()


def _scs_fn(w_hbm, out_hbm, sbuf0, sbuf1, ssin0, ssin1, ssout0, ssout1,
            vbuf0, vbuf1, vsin0, vsin1, vsout0, vsout1):
    del vbuf0, vbuf1, vsin0, vsin1, vsout0, vsout1
    base = lax.axis_index("c") * ROWS_PER_CORE
    _pipelined_copy(w_hbm, out_hbm, base, SCS_CHUNK, SCS_NCHUNK,
                    (sbuf0, sbuf1), (ssin0, ssin1), (ssout0, ssout1))


def _tec_fn(w_hbm, out_hbm, sbuf0, sbuf1, ssin0, ssin1, ssout0, ssout1,
            vbuf0, vbuf1, vsin0, vsin1, vsout0, vsout1):
    del sbuf0, sbuf1, ssin0, ssin1, ssout0, ssout1
    base = (lax.axis_index("c") * ROWS_PER_CORE + SCS_ROWS
            + lax.axis_index("s") * TEC_ROWS)
    _pipelined_copy(w_hbm, out_hbm, base, TEC_CHUNK, TEC_NCHUNK,
                    (vbuf0, vbuf1), (vsin0, vsin1), (vsout0, vsout1))


_sc_copy = mpmd.mpmd_map(
    [(_scalar_mesh, _scs_fn), (_vector_mesh, _tec_fn)],
    out_types=jax.ShapeDtypeStruct((MAX_LEN, D_MODEL), jnp.float32),
    scratch_types=[
        pltpu.VMEM_SHARED((SCS_CHUNK, D_MODEL), jnp.float32),
        pltpu.VMEM_SHARED((SCS_CHUNK, D_MODEL), jnp.float32),
        pltpu.SemaphoreType.DMA @ _scalar_mesh,
        pltpu.SemaphoreType.DMA @ _scalar_mesh,
        pltpu.SemaphoreType.DMA @ _scalar_mesh,
        pltpu.SemaphoreType.DMA @ _scalar_mesh,
        (_MS.VMEM @ _vector_mesh)((TEC_CHUNK, D_MODEL), jnp.float32),
        (_MS.VMEM @ _vector_mesh)((TEC_CHUNK, D_MODEL), jnp.float32),
        pltpu.SemaphoreType.DMA @ _vector_mesh,
        pltpu.SemaphoreType.DMA @ _vector_mesh,
        pltpu.SemaphoreType.DMA @ _vector_mesh,
        pltpu.SemaphoreType.DMA @ _vector_mesh,
    ],
)


def kernel(input_ids, W):
    del input_ids
    return _sc_copy(W)[None]


# speedup vs baseline: 1.0154x; 1.0154x over previous
import functools

import jax
import jax.numpy as jnp
from jax import lax
from jax.experimental import pallas as pl
from jax.experimental.pallas import tpu as pltpu
from jax.experimental.pallas import tpu_sc as plsc
from jax._src.pallas import mpmd
from jax._src.pallas.mosaic.core import MemorySpace as _MS

MAX_LEN = 8192
D_MODEL = 1024
NUM_CORES = 2
NUM_SUBCORES = 16
ROWS_PER_CORE = MAX_LEN // NUM_CORES

SCS_ROWS = 1536
SCS_CHUNK = 128
SCS_NCHUNK = SCS_ROWS // SCS_CHUNK

TEC_ROWS_PER_CORE = ROWS_PER_CORE - SCS_ROWS
TEC_ROWS = TEC_ROWS_PER_CORE // NUM_SUBCORES
TEC_CHUNK = 32
TEC_NCHUNK = TEC_ROWS // TEC_CHUNK

_scalar_mesh = plsc.ScalarSubcoreMesh(axis_name="c", num_cores=NUM_CORES)
_vector_mesh = plsc.VectorSubcoreMesh(core_axis_name="c", subcore_axis_name="s")


def _pipelined_copy(w_hbm, out_hbm, base, chunk, nchunk, bufs, sins, souts):
    def in_slice(c):
        return w_hbm.at[pl.ds(base + c * chunk, chunk)]

    def out_slice(c):
        return out_hbm.at[pl.ds(base + c * chunk, chunk)]

    hin = [None] * nchunk
    hout = [None] * nchunk
    hin[0] = pltpu.async_copy(in_slice(0), bufs[0], sins[0])
    for c in range(nchunk):
        if c + 1 < nchunk:
            b = (c + 1) % 2
            if c >= 1:
                hout[c - 1].wait()
            hin[c + 1] = pltpu.async_copy(in_slice(c + 1), bufs[b], sins[b])
        hin[c].wait()
        hout[c] = pltpu.async_copy(bufs[c % 2], out_slice(c), souts[c % 2])
    if nchunk >= 2:
        hout[nchunk - 2].wait()
    hout[nchunk - 1].wait()


def _scs_fn(w_hbm, out_hbm, sbuf0, sbuf1, ssin0, ssin1, ssout0, ssout1,
            vbuf0, vbuf1, vsin0, vsin1, vsout0, vsout1):
    del vbuf0, vbuf1, vsin0, vsin1, vsout0, vsout1
    base = lax.axis_index("c") * ROWS_PER_CORE
    _pipelined_copy(w_hbm, out_hbm, base, SCS_CHUNK, SCS_NCHUNK,
                    (sbuf0, sbuf1), (ssin0, ssin1), (ssout0, ssout1))


def _tec_fn(w_hbm, out_hbm, sbuf0, sbuf1, ssin0, ssin1, ssout0, ssout1,
            vbuf0, vbuf1, vsin0, vsin1, vsout0, vsout1):
    del sbuf0, sbuf1, ssin0, ssin1, ssout0, ssout1
    base = (lax.axis_index("c") * ROWS_PER_CORE + SCS_ROWS
            + lax.axis_index("s") * TEC_ROWS)
    _pipelined_copy(w_hbm, out_hbm, base, TEC_CHUNK, TEC_NCHUNK,
                    (vbuf0, vbuf1), (vsin0, vsin1), (vsout0, vsout1))


_sc_copy = mpmd.mpmd_map(
    [(_scalar_mesh, _scs_fn), (_vector_mesh, _tec_fn)],
    out_types=jax.ShapeDtypeStruct((MAX_LEN, D_MODEL), jnp.float32),
    scratch_types=[
        pltpu.VMEM_SHARED((SCS_CHUNK, D_MODEL), jnp.float32),
        pltpu.VMEM_SHARED((SCS_CHUNK, D_MODEL), jnp.float32),
        pltpu.SemaphoreType.DMA @ _scalar_mesh,
        pltpu.SemaphoreType.DMA @ _scalar_mesh,
        pltpu.SemaphoreType.DMA @ _scalar_mesh,
        pltpu.SemaphoreType.DMA @ _scalar_mesh,
        (_MS.VMEM @ _vector_mesh)((TEC_CHUNK, D_MODEL), jnp.float32),
        (_MS.VMEM @ _vector_mesh)((TEC_CHUNK, D_MODEL), jnp.float32),
        pltpu.SemaphoreType.DMA @ _vector_mesh,
        pltpu.SemaphoreType.DMA @ _vector_mesh,
        pltpu.SemaphoreType.DMA @ _vector_mesh,
        pltpu.SemaphoreType.DMA @ _vector_mesh,
    ],
)


def kernel(input_ids, W):
    del input_ids
    return _sc_copy(W)[None]


# speedup vs baseline: 1.0963x; 1.0797x over previous
import functools

import jax
import jax.numpy as jnp
from jax import lax
from jax.experimental import pallas as pl
from jax.experimental.pallas import tpu as pltpu
from jax.experimental.pallas import tpu_sc as plsc
from jax._src.pallas import mpmd
from jax._src.pallas.mosaic.core import MemorySpace as _MS

MAX_LEN = 8192
D_MODEL = 1024
NUM_CORES = 2
NUM_SUBCORES = 16
ROWS_PER_CORE = MAX_LEN // NUM_CORES

SCS_ROWS = 1536
SCS_CHUNK = 256
SCS_NCHUNK = SCS_ROWS // SCS_CHUNK

TEC_ROWS_PER_CORE = ROWS_PER_CORE - SCS_ROWS
TEC_ROWS = TEC_ROWS_PER_CORE // NUM_SUBCORES
TEC_CHUNK = 32
TEC_NCHUNK = TEC_ROWS // TEC_CHUNK

_scalar_mesh = plsc.ScalarSubcoreMesh(axis_name="c", num_cores=NUM_CORES)
_vector_mesh = plsc.VectorSubcoreMesh(core_axis_name="c", subcore_axis_name="s")


def _pipelined_copy(w_hbm, out_hbm, base, chunk, nchunk, bufs, sins, souts):
    def in_slice(c):
        return w_hbm.at[pl.ds(base + c * chunk, chunk)]

    def out_slice(c):
        return out_hbm.at[pl.ds(base + c * chunk, chunk)]

    hin = [None] * nchunk
    hout = [None] * nchunk
    hin[0] = pltpu.async_copy(in_slice(0), bufs[0], sins[0])
    for c in range(nchunk):
        if c + 1 < nchunk:
            b = (c + 1) % 2
            if c >= 1:
                hout[c - 1].wait()
            hin[c + 1] = pltpu.async_copy(in_slice(c + 1), bufs[b], sins[b])
        hin[c].wait()
        hout[c] = pltpu.async_copy(bufs[c % 2], out_slice(c), souts[c % 2])
    if nchunk >= 2:
        hout[nchunk - 2].wait()
    hout[nchunk - 1].wait()


def _scs_fn(w_hbm, out_hbm, sbuf0, sbuf1, ssin0, ssin1, ssout0, ssout1,
            vbuf0, vbuf1, vsin0, vsin1, vsout0, vsout1):
    del vbuf0, vbuf1, vsin0, vsin1, vsout0, vsout1
    base = lax.axis_index("c") * ROWS_PER_CORE
    _pipelined_copy(w_hbm, out_hbm, base, SCS_CHUNK, SCS_NCHUNK,
                    (sbuf0, sbuf1), (ssin0, ssin1), (ssout0, ssout1))


def _tec_fn(w_hbm, out_hbm, sbuf0, sbuf1, ssin0, ssin1, ssout0, ssout1,
            vbuf0, vbuf1, vsin0, vsin1, vsout0, vsout1):
    del sbuf0, sbuf1, ssin0, ssin1, ssout0, ssout1
    base = (lax.axis_index("c") * ROWS_PER_CORE + SCS_ROWS
            + lax.axis_index("s") * TEC_ROWS)
    _pipelined_copy(w_hbm, out_hbm, base, TEC_CHUNK, TEC_NCHUNK,
                    (vbuf0, vbuf1), (vsin0, vsin1), (vsout0, vsout1))


_sc_copy = mpmd.mpmd_map(
    [(_scalar_mesh, _scs_fn), (_vector_mesh, _tec_fn)],
    out_types=jax.ShapeDtypeStruct((MAX_LEN, D_MODEL), jnp.float32),
    scratch_types=[
        pltpu.VMEM_SHARED((SCS_CHUNK, D_MODEL), jnp.float32),
        pltpu.VMEM_SHARED((SCS_CHUNK, D_MODEL), jnp.float32),
        pltpu.SemaphoreType.DMA @ _scalar_mesh,
        pltpu.SemaphoreType.DMA @ _scalar_mesh,
        pltpu.SemaphoreType.DMA @ _scalar_mesh,
        pltpu.SemaphoreType.DMA @ _scalar_mesh,
        (_MS.VMEM @ _vector_mesh)((TEC_CHUNK, D_MODEL), jnp.float32),
        (_MS.VMEM @ _vector_mesh)((TEC_CHUNK, D_MODEL), jnp.float32),
        pltpu.SemaphoreType.DMA @ _vector_mesh,
        pltpu.SemaphoreType.DMA @ _vector_mesh,
        pltpu.SemaphoreType.DMA @ _vector_mesh,
        pltpu.SemaphoreType.DMA @ _vector_mesh,
    ],
)


def kernel(input_ids, W):
    del input_ids
    return _sc_copy(W)[None]
